# baseline (speedup 1.0000x reference)
import functools

import jax
import jax.numpy as jnp
from jax import lax
from jax.experimental import pallas as pl
from jax.experimental.pallas import tpu as pltpu
from jax.experimental.pallas import tpu_sc as plsc

N_NODES = 10000
N_EDGES = 320000
D = 128
N_GRAPHS = 64
NEG_SLOPE = (1.0 / 8.0 + 1.0 / 3.0) / 2.0

NC = 2
NS = 16
NW = NC * NS
CHUNK = 125
N_CHUNK = 80
EPT = CHUNK * N_CHUNK
NBUF = 2
HC = N_CHUNK // 2
ACC_ROWS = 10240
TROWS = ACC_ROWS // NS
ZROWS = 80
TAIL_ROWS = N_NODES - (NS - 1) * TROWS


def _gwait(x_hbm, idx_s, rows, sem, c, b):
    pltpu.make_async_copy(x_hbm.at[idx_s.at[c]], rows.at[b], sem).wait()


def _swait(acc, idx_d, rows, sem_s, c, b):
    pltpu.make_async_copy(rows.at[b], acc.at[idx_d.at[c]], sem_s).wait()


def _agg_body(x_hbm, src_hbm, dst_hbm, out_hbm,
              idx_s, idx_d, rows, acc, sem, sem_s):
    cid = lax.axis_index("c")
    sid = lax.axis_index("s")
    wid = sid * NC + cid

    zvec = jnp.zeros((16,), jnp.float32)

    def zrow(i, _):
        for k in range(D // 16):
            rows[0, i, pl.ds(k * 16, 16)] = zvec
        return 0

    lax.fori_loop(0, ZROWS, zrow, 0)
    trow0 = pl.multiple_of(sid * TROWS, 8)
    for r in range(TROWS // ZROWS):
        pltpu.sync_copy(rows.at[0, pl.ds(0, ZROWS)],
                        acc.at[pl.ds(trow0 + r * ZROWS, ZROWS)])
    plsc.subcore_barrier()

    for h in range(2):
        pltpu.sync_copy(src_hbm.at[wid, pl.ds(h * HC, HC)], idx_s)
        pltpu.sync_copy(dst_hbm.at[wid, pl.ds(h * HC, HC)], idx_d)
        pltpu.async_copy(x_hbm.at[idx_s.at[0]], rows.at[0], sem)
        _gwait(x_hbm, idx_s, rows, sem, 0, 0)
        pltpu.async_copy(rows.at[0], acc.at[idx_d.at[0]], sem_s, add=True)
        pltpu.async_copy(x_hbm.at[idx_s.at[1]], rows.at[1], sem)

        def outer(k, _):
            c0 = 2 * k + 1
            _gwait(x_hbm, idx_s, rows, sem, c0, 1)
            pltpu.async_copy(rows.at[1], acc.at[idx_d.at[c0]], sem_s,
                             add=True)
            _swait(acc, idx_d, rows, sem_s, c0 - 1, 0)
            pltpu.async_copy(x_hbm.at[idx_s.at[c0 + 1]], rows.at[0], sem)
            c1 = c0 + 1
            _gwait(x_hbm, idx_s, rows, sem, c1, 0)
            pltpu.async_copy(rows.at[0], acc.at[idx_d.at[c1]], sem_s,
                             add=True)
            _swait(acc, idx_d, rows, sem_s, c1 - 1, 1)
            pltpu.async_copy(x_hbm.at[idx_s.at[c1 + 1]], rows.at[1], sem)
            return 0

        lax.fori_loop(0, (HC - 2) // 2, outer, 0)
        _gwait(x_hbm, idx_s, rows, sem, HC - 1, 1)
        pltpu.async_copy(rows.at[1], acc.at[idx_d.at[HC - 1]], sem_s,
                         add=True)
        _swait(acc, idx_d, rows, sem_s, HC - 2, 0)
        _swait(acc, idx_d, rows, sem_s, HC - 1, 1)
    plsc.subcore_barrier()

    @pl.when(sid < NS - 1)
    def _drain_main():
        pltpu.sync_copy(acc.at[pl.ds(trow0, TROWS)],
                        out_hbm.at[cid, pl.ds(trow0, TROWS)])

    @pl.when(sid == NS - 1)
    def _drain_tail():
        pltpu.sync_copy(acc.at[pl.ds(trow0, TAIL_ROWS)],
                        out_hbm.at[cid, pl.ds(trow0, TAIL_ROWS)])


@functools.cache
def _make_agg():
    return pl.kernel(
        _agg_body,
        out_type=jax.ShapeDtypeStruct((NC, N_NODES, D), jnp.float32),
        mesh=plsc.VectorSubcoreMesh(core_axis_name="c", subcore_axis_name="s",
                                    num_cores=NC, num_subcores=NS),
        scratch_types=[
            pltpu.VMEM((HC, CHUNK), jnp.int32),
            pltpu.VMEM((HC, CHUNK), jnp.int32),
            pltpu.VMEM((NBUF, CHUNK, D), jnp.float32),
            pltpu.VMEM_SHARED((ACC_ROWS, D), jnp.float32),
            pltpu.SemaphoreType.DMA,
            pltpu.SemaphoreType.DMA,
        ],
    )


def _agg(x, src3, dst3):
    return _make_agg()(x, src3, dst3)


ROW_BLK = 1000


def _mlp_body(x_ref, parts_ref, w1_ref, b1_ref, w2_ref, b2_ref, o_ref):
    h = x_ref[...] + parts_ref[0] + parts_ref[1]
    h = jnp.dot(h, w1_ref[...], preferred_element_type=jnp.float32)
    h = jnp.maximum(h + b1_ref[...], 0.0)
    h = jnp.dot(h, w2_ref[...], preferred_element_type=jnp.float32)
    h = h + b2_ref[...]
    o_ref[...] = jnp.where(h >= 0, h, h * NEG_SLOPE)


def _mlp(x, parts, w1, b1, w2, b2):
    return pl.pallas_call(
        _mlp_body,
        grid=(N_NODES // ROW_BLK,),
        in_specs=[
            pl.BlockSpec((ROW_BLK, D), lambda i: (i, 0)),
            pl.BlockSpec((NC, ROW_BLK, D), lambda i: (0, i, 0)),
            pl.BlockSpec((D, D), lambda i: (0, 0)),
            pl.BlockSpec((1, D), lambda i: (0, 0)),
            pl.BlockSpec((D, D), lambda i: (0, 0)),
            pl.BlockSpec((1, D), lambda i: (0, 0)),
        ],
        out_specs=pl.BlockSpec((ROW_BLK, D), lambda i: (i, 0)),
        out_shape=jax.ShapeDtypeStruct((N_NODES, D), jnp.float32),
    )(x, parts, w1, b1, w2, b2)


def _pool_body(x_ref, batch_ref, lin_w_ref, lin_b_ref, ln_g_ref, ln_b_ref,
               o_ref):
    xv = x_ref[...]
    b = batch_ref[...]
    gids = lax.broadcasted_iota(jnp.int32, (1, N_GRAPHS), 1)
    oh = (b == gids).astype(jnp.float32)
    pooled = lax.dot_general(oh, xv, (((0,), (0,)), ((), ())))
    ones = jnp.ones((N_NODES, 1), jnp.float32)
    counts = lax.dot_general(oh, ones, (((0,), (0,)), ((), ())))
    y = jnp.dot(pooled, lin_w_ref[...], preferred_element_type=jnp.float32)
    y = y + counts * lin_b_ref[...]
    mu = jnp.mean(y, axis=1, keepdims=True)
    var = jnp.mean((y - mu) ** 2, axis=1, keepdims=True)
    o_ref[...] = (y - mu) * lax.rsqrt(var + 1e-5) * ln_g_ref[...] + ln_b_ref[...]


def _pool(x, batch2d, lin_w, lin_b, ln_g, ln_b):
    return pl.pallas_call(
        _pool_body,
        out_shape=jax.ShapeDtypeStruct((N_GRAPHS, D), jnp.float32),
    )(x, batch2d, lin_w, lin_b, ln_g, ln_b)


def kernel(x, edge_index, batch, g1w1, g1b1, g1w2, g1b2, g2w1, g2b1, g2w2,
           g2b2, g3w1, g3b1, g3w2, g3b2, g4w1, g4b1, g4w2, g4b2, g5w1, g5b1,
           g5w2, g5b2, lin_w, lin_b, ln_g, ln_b):
    src = edge_index[0].astype(jnp.int32)
    dst = edge_index[1].astype(jnp.int32)
    src3 = src.reshape(NW, N_CHUNK, CHUNK)
    dst3 = dst.reshape(NW, N_CHUNK, CHUNK)
    layers = [
        (g1w1, g1b1, g1w2, g1b2),
        (g2w1, g2b1, g2w2, g2b2),
        (g3w1, g3b1, g3w2, g3b2),
        (g4w1, g4b1, g4w2, g4b2),
        (g5w1, g5b1, g5w2, g5b2),
    ]
    for w1, b1, w2, b2 in layers:
        parts = _agg(x, src3, dst3)
        x = _mlp(x, parts, w1, b1.reshape(1, D), w2, b2.reshape(1, D))
    return _pool(x, batch.astype(jnp.int32).reshape(N_NODES, 1), lin_w,
                 lin_b.reshape(1, D), ln_g.reshape(1, D), ln_b.reshape(1, D))


# speedup vs baseline: 1.2116x; 1.2116x over previous
import functools

import jax
import jax.numpy as jnp
from jax import lax
from jax.experimental import pallas as pl
from jax.experimental.pallas import tpu as pltpu
from jax.experimental.pallas import tpu_sc as plsc

N_NODES = 10000
N_EDGES = 320000
D = 128
N_GRAPHS = 64
NEG_SLOPE = (1.0 / 8.0 + 1.0 / 3.0) / 2.0

NC = 2
NS = 16
NW = NC * NS
CHUNK = 125
N_CHUNK = 80
EPT = CHUNK * N_CHUNK
NBUF = 2
HC = N_CHUNK // 2
ACC_ROWS = 10240
TROWS = ACC_ROWS // NS
ZROWS = 80
TAIL_ROWS = N_NODES - (NS - 1) * TROWS


def _agg_body(x_hbm, src_hbm, dst_hbm, out_hbm,
              idx_s, idx_d, rows, acc, sem):
    cid = lax.axis_index("c")
    sid = lax.axis_index("s")
    wid = sid * NC + cid

    pltpu.sync_copy(src_hbm.at[wid, pl.ds(0, HC)], idx_s)
    pltpu.sync_copy(dst_hbm.at[wid, pl.ds(0, HC)], idx_d)
    zvec = jnp.zeros((16,), jnp.float32)

    def zrow(i, _):
        for k in range(D // 16):
            rows[0, i, pl.ds(k * 16, 16)] = zvec
        return 0

    lax.fori_loop(0, ZROWS, zrow, 0)
    trow0 = pl.multiple_of(sid * TROWS, 8)
    for r in range(TROWS // ZROWS):
        pltpu.sync_copy(rows.at[0, pl.ds(0, ZROWS)],
                        acc.at[pl.ds(trow0 + r * ZROWS, ZROWS)])
    for b in range(NBUF):
        pltpu.async_copy(x_hbm.at[idx_s.at[b]], rows.at[b], sem)
    plsc.subcore_barrier()

    for h in range(2):
        if h == 1:
            pltpu.sync_copy(src_hbm.at[wid, pl.ds(HC, HC)], idx_s)
            pltpu.sync_copy(dst_hbm.at[wid, pl.ds(HC, HC)], idx_d)
            for b in range(NBUF):
                pltpu.async_copy(x_hbm.at[idx_s.at[b]], rows.at[b], sem)

        def outer(g, _):
            for b in range(NBUF):
                lc = g * NBUF + b
                pltpu.make_async_copy(x_hbm.at[idx_s.at[lc]], rows.at[b],
                                      sem).wait()
                pltpu.sync_copy(rows.at[b], acc.at[idx_d.at[lc]], add=True)
                pltpu.async_copy(x_hbm.at[idx_s.at[lc + NBUF]], rows.at[b],
                                 sem)
            return 0

        lax.fori_loop(0, HC // NBUF - 1, outer, 0)
        for b in range(NBUF):
            lc = HC - NBUF + b
            pltpu.make_async_copy(x_hbm.at[idx_s.at[lc]], rows.at[b],
                                  sem).wait()
            pltpu.sync_copy(rows.at[b], acc.at[idx_d.at[lc]], add=True)
    plsc.subcore_barrier()

    @pl.when(sid < NS - 1)
    def _drain_main():
        pltpu.sync_copy(acc.at[pl.ds(trow0, TROWS)],
                        out_hbm.at[cid, pl.ds(trow0, TROWS)])

    @pl.when(sid == NS - 1)
    def _drain_tail():
        pltpu.sync_copy(acc.at[pl.ds(trow0, TAIL_ROWS)],
                        out_hbm.at[cid, pl.ds(trow0, TAIL_ROWS)])


@functools.cache
def _make_agg():
    return pl.kernel(
        _agg_body,
        out_type=jax.ShapeDtypeStruct((NC, N_NODES, D), jnp.float32),
        mesh=plsc.VectorSubcoreMesh(core_axis_name="c", subcore_axis_name="s",
                                    num_cores=NC, num_subcores=NS),
        scratch_types=[
            pltpu.VMEM((HC, CHUNK), jnp.int32),
            pltpu.VMEM((HC, CHUNK), jnp.int32),
            pltpu.VMEM((NBUF, CHUNK, D), jnp.float32),
            pltpu.VMEM_SHARED((ACC_ROWS, D), jnp.float32),
            pltpu.SemaphoreType.DMA,
        ],
    )


def _agg(x, src3, dst3):
    return _make_agg()(x, src3, dst3)


ROW_BLK = 2000


def _mlp_body(x_ref, parts_ref, w1_ref, b1_ref, w2_ref, b2_ref, o_ref):
    h = x_ref[...] + parts_ref[0] + parts_ref[1]
    h = jnp.dot(h, w1_ref[...], preferred_element_type=jnp.float32)
    h = jnp.maximum(h + b1_ref[...], 0.0)
    h = jnp.dot(h, w2_ref[...], preferred_element_type=jnp.float32)
    h = h + b2_ref[...]
    o_ref[...] = jnp.where(h >= 0, h, h * NEG_SLOPE)


def _mlp(x, parts, w1, b1, w2, b2):
    return pl.pallas_call(
        _mlp_body,
        grid=(N_NODES // ROW_BLK,),
        in_specs=[
            pl.BlockSpec((ROW_BLK, D), lambda i: (i, 0)),
            pl.BlockSpec((NC, ROW_BLK, D), lambda i: (0, i, 0)),
            pl.BlockSpec((D, D), lambda i: (0, 0)),
            pl.BlockSpec((1, D), lambda i: (0, 0)),
            pl.BlockSpec((D, D), lambda i: (0, 0)),
            pl.BlockSpec((1, D), lambda i: (0, 0)),
        ],
        out_specs=pl.BlockSpec((ROW_BLK, D), lambda i: (i, 0)),
        out_shape=jax.ShapeDtypeStruct((N_NODES, D), jnp.float32),
    )(x, parts, w1, b1, w2, b2)


def _pool_body(x_ref, batch_ref, lin_w_ref, lin_b_ref, ln_g_ref, ln_b_ref,
               o_ref):
    xv = x_ref[...]
    b = batch_ref[...]
    gids = lax.broadcasted_iota(jnp.int32, (1, N_GRAPHS), 1)
    oh = (b == gids).astype(jnp.float32)
    pooled = lax.dot_general(oh, xv, (((0,), (0,)), ((), ())))
    ones = jnp.ones((N_NODES, 1), jnp.float32)
    counts = lax.dot_general(oh, ones, (((0,), (0,)), ((), ())))
    y = jnp.dot(pooled, lin_w_ref[...], preferred_element_type=jnp.float32)
    y = y + counts * lin_b_ref[...]
    mu = jnp.mean(y, axis=1, keepdims=True)
    var = jnp.mean((y - mu) ** 2, axis=1, keepdims=True)
    o_ref[...] = (y - mu) * lax.rsqrt(var + 1e-5) * ln_g_ref[...] + ln_b_ref[...]


def _pool(x, batch2d, lin_w, lin_b, ln_g, ln_b):
    return pl.pallas_call(
        _pool_body,
        out_shape=jax.ShapeDtypeStruct((N_GRAPHS, D), jnp.float32),
    )(x, batch2d, lin_w, lin_b, ln_g, ln_b)


def kernel(x, edge_index, batch, g1w1, g1b1, g1w2, g1b2, g2w1, g2b1, g2w2,
           g2b2, g3w1, g3b1, g3w2, g3b2, g4w1, g4b1, g4w2, g4b2, g5w1, g5b1,
           g5w2, g5b2, lin_w, lin_b, ln_g, ln_b):
    src = edge_index[0].astype(jnp.int32)
    dst = edge_index[1].astype(jnp.int32)
    src3 = src.reshape(NW, N_CHUNK, CHUNK)
    dst3 = dst.reshape(NW, N_CHUNK, CHUNK)
    layers = [
        (g1w1, g1b1, g1w2, g1b2),
        (g2w1, g2b1, g2w2, g2b2),
        (g3w1, g3b1, g3w2, g3b2),
        (g4w1, g4b1, g4w2, g4b2),
        (g5w1, g5b1, g5w2, g5b2),
    ]
    for w1, b1, w2, b2 in layers:
        parts = _agg(x, src3, dst3)
        x = _mlp(x, parts, w1, b1.reshape(1, D), w2, b2.reshape(1, D))
    return _pool(x, batch.astype(jnp.int32).reshape(N_NODES, 1), lin_w,
                 lin_b.reshape(1, D), ln_g.reshape(1, D), ln_b.reshape(1, D))
